# baseline (device time: 85043 ns/iter reference)
import jax
import jax.numpy as jnp
from jax import lax
from jax.experimental import pallas as pl
from jax.experimental.pallas import tpu as pltpu

N_DEV = 4
BLK = 64
SCALE = 0.125
NEG = -1e9


def kernel(x, Wq, K_ext, V_ext, Wo):
    B, Sq, Dm = x.shape
    _, Skv_l, Hq, Dh = K_ext.shape
    Hl = Wq.shape[1] // Dh
    Do = Wo.shape[1]

    def body(x_ref, wq_ref, k_ref, v_ref, wo_ref, out_ref,
             kbuf, vbuf, part, pbuf,
             ksend, vsend, krecv, vrecv, psend, precv):
        my = lax.axis_index("i")

        barrier = pltpu.get_barrier_semaphore()
        for j in range(1, N_DEV):
            pl.semaphore_signal(
                barrier, inc=1,
                device_id=((my + j) % N_DEV,),
                device_id_type=pl.DeviceIdType.MESH,
            )
        pl.semaphore_wait(barrier, N_DEV - 1)

        kv_rdmas = []
        for j in range(1, N_DEV):
            kv_rdmas.append(pltpu.make_async_remote_copy(
                src_ref=k_ref.at[:, :, pl.ds(j * Hl, Hl), :],
                dst_ref=kbuf,
                send_sem=ksend.at[j - 1],
                recv_sem=krecv,
                device_id=(j,),
                device_id_type=pl.DeviceIdType.MESH,
            ))
            kv_rdmas.append(pltpu.make_async_remote_copy(
                src_ref=v_ref.at[:, :, pl.ds(j * Hl, Hl), :],
                dst_ref=vbuf,
                send_sem=vsend.at[j - 1],
                recv_sem=vrecv,
                device_id=(j,),
                device_id_type=pl.DeviceIdType.MESH,
            ))

        @pl.when(my == 0)
        def _():
            for r in kv_rdmas:
                r.start()
            kbuf[...] = k_ref[:, :, pl.ds(0, Hl), :]
            vbuf[...] = v_ref[:, :, pl.ds(0, Hl), :]

        q = [
            lax.dot_general(
                x_ref[b], wq_ref[...],
                (((1,), (0,)), ((), ())),
                preferred_element_type=jnp.float32,
            )
            for b in range(B)
        ]

        @pl.when(my != 0)
        def _():
            kv_rdmas[0].wait_recv()
            kv_rdmas[1].wait_recv()

        rb = lax.broadcasted_iota(jnp.int32, (Sq, Skv_l), 0) // BLK
        cb = lax.broadcasted_iota(jnp.int32, (Sq, Skv_l), 1) // BLK
        mask = rb >= cb
        wo = wo_ref[...]
        for b in range(B):
            kb_all = kbuf[b]
            vb_all = vbuf[b]
            acc = jnp.zeros((Sq, Do), jnp.float32)
            for h in range(Hl):
                qh = q[b][:, h * Dh:(h + 1) * Dh]
                kh = kb_all[:, h, :]
                vh = vb_all[:, h, :]
                s = lax.dot_general(
                    qh, kh, (((1,), (1,)), ((), ())),
                    preferred_element_type=jnp.float32,
                ) * SCALE
                s = jnp.where(mask, s, NEG)
                m = jnp.max(s, axis=1, keepdims=True)
                e = jnp.exp(s - m)
                w = e / jnp.sum(e, axis=1, keepdims=True)
                ctx = lax.dot_general(
                    w, vh, (((1,), (0,)), ((), ())),
                    preferred_element_type=jnp.float32,
                )
                acc = acc + lax.dot_general(
                    ctx, wo[h * Dh:(h + 1) * Dh, :],
                    (((1,), (0,)), ((), ())),
                    preferred_element_type=jnp.float32,
                )
            part[b, :, :] = acc

        p_rdmas = []
        for d in range(1, N_DEV):
            p_rdmas.append(pltpu.make_async_remote_copy(
                src_ref=part,
                dst_ref=pbuf.at[d - 1],
                send_sem=psend.at[d - 1],
                recv_sem=precv.at[d - 1],
                device_id=((my + d) % N_DEV,),
                device_id_type=pl.DeviceIdType.MESH,
            ))
        for r in p_rdmas:
            r.start()
        for r in p_rdmas:
            r.wait_recv()
        for b in range(B):
            out_ref[b, :, :] = (
                part[b] + pbuf[0, b] + pbuf[1, b] + pbuf[2, b]
            )
        for r in p_rdmas:
            r.wait_send()

        @pl.when(my == 0)
        def _():
            for r in kv_rdmas:
                r.wait_send()

    return pl.pallas_call(
        body,
        out_shape=jax.ShapeDtypeStruct((B, Sq, Do), jnp.float32),
        in_specs=[pl.BlockSpec(memory_space=pltpu.VMEM)] * 5,
        out_specs=pl.BlockSpec(memory_space=pltpu.VMEM),
        scratch_shapes=[
            pltpu.VMEM((B, Skv_l, Hl, Dh), jnp.float32),
            pltpu.VMEM((B, Skv_l, Hl, Dh), jnp.float32),
            pltpu.VMEM((B, Sq, Do), jnp.float32),
            pltpu.VMEM((N_DEV - 1, B, Sq, Do), jnp.float32),
            pltpu.SemaphoreType.DMA((N_DEV - 1,)),
            pltpu.SemaphoreType.DMA((N_DEV - 1,)),
            pltpu.SemaphoreType.DMA,
            pltpu.SemaphoreType.DMA,
            pltpu.SemaphoreType.DMA((N_DEV - 1,)),
            pltpu.SemaphoreType.DMA((N_DEV - 1,)),
        ],
        compiler_params=pltpu.CompilerParams(collective_id=0),
    )(x, Wq, K_ext, V_ext, Wo)


# device time: 82012 ns/iter; 1.0370x vs baseline; 1.0370x over previous
import jax
import jax.numpy as jnp
from jax import lax
from jax.experimental import pallas as pl
from jax.experimental.pallas import tpu as pltpu

N_DEV = 4
BLK = 64
SCALE = 0.125
NEG = -1e9


def kernel(x, Wq, K_ext, V_ext, Wo):
    B, Sq, Dm = x.shape
    _, Skv_l, Hq, Dh = K_ext.shape
    Hl = Wq.shape[1] // Dh
    Do = Wo.shape[1]

    def body(x_ref, wq_ref, k_ref, v_ref, wo_ref, out_ref,
             kbuf, vbuf, part, rbuf, sbuf, tbuf,
             ksend, vsend, krecv, vrecv, psend, precv):
        my = lax.axis_index("i")

        q = [
            lax.dot_general(
                x_ref[b], wq_ref[...],
                (((1,), (0,)), ((), ())),
                preferred_element_type=jnp.float32,
            )
            for b in range(B)
        ]

        barrier = pltpu.get_barrier_semaphore()
        for j in range(1, N_DEV):
            pl.semaphore_signal(
                barrier, inc=1,
                device_id=((my + j) % N_DEV,),
                device_id_type=pl.DeviceIdType.MESH,
            )
        pl.semaphore_wait(barrier, N_DEV - 1)

        kv_rdmas = []
        for j in (2, 1, 3):
            kv_rdmas.append(pltpu.make_async_remote_copy(
                src_ref=k_ref.at[:, :, pl.ds(j * Hl, Hl), :],
                dst_ref=kbuf,
                send_sem=ksend.at[j - 1],
                recv_sem=krecv,
                device_id=(j,),
                device_id_type=pl.DeviceIdType.MESH,
            ))
            kv_rdmas.append(pltpu.make_async_remote_copy(
                src_ref=v_ref.at[:, :, pl.ds(j * Hl, Hl), :],
                dst_ref=vbuf,
                send_sem=vsend.at[j - 1],
                recv_sem=vrecv,
                device_id=(j,),
                device_id_type=pl.DeviceIdType.MESH,
            ))

        @pl.when(my == 0)
        def _():
            for r in kv_rdmas:
                r.start()
            kbuf[...] = k_ref[:, :, pl.ds(0, Hl), :]
            vbuf[...] = v_ref[:, :, pl.ds(0, Hl), :]

        @pl.when(my != 0)
        def _():
            kv_rdmas[0].wait_recv()
            kv_rdmas[1].wait_recv()

        rb = lax.broadcasted_iota(jnp.int32, (Sq, Skv_l), 0) // BLK
        cb = lax.broadcasted_iota(jnp.int32, (Sq, Skv_l), 1) // BLK
        mask = rb >= cb
        wo = wo_ref[...]
        for b in range(B):
            kb_all = kbuf[b]
            vb_all = vbuf[b]
            acc = jnp.zeros((Sq, Do), jnp.float32)
            for h in range(Hl):
                qh = q[b][:, h * Dh:(h + 1) * Dh]
                kh = kb_all[:, h, :]
                vh = vb_all[:, h, :]
                s = lax.dot_general(
                    qh, kh, (((1,), (1,)), ((), ())),
                    preferred_element_type=jnp.float32,
                ) * SCALE
                s = jnp.where(mask, s, NEG)
                m = jnp.max(s, axis=1, keepdims=True)
                e = jnp.exp(s - m)
                w = e / jnp.sum(e, axis=1, keepdims=True)
                ctx = lax.dot_general(
                    w, vh, (((1,), (0,)), ((), ())),
                    preferred_element_type=jnp.float32,
                )
                acc = acc + lax.dot_general(
                    ctx, wo[h * Dh:(h + 1) * Dh, :],
                    (((1,), (0,)), ((), ())),
                    preferred_element_type=jnp.float32,
                )
            part[b, :, :] = acc

        keep0 = my // 2 == 0
        p1 = 3 - my
        p2 = my ^ 1

        ex1a = pltpu.make_async_remote_copy(
            src_ref=part.at[1], dst_ref=rbuf.at[0],
            send_sem=psend.at[0], recv_sem=precv.at[0],
            device_id=(p1,), device_id_type=pl.DeviceIdType.MESH,
        )
        ex1b = pltpu.make_async_remote_copy(
            src_ref=part.at[0], dst_ref=rbuf.at[0],
            send_sem=psend.at[0], recv_sem=precv.at[0],
            device_id=(p1,), device_id_type=pl.DeviceIdType.MESH,
        )

        @pl.when(keep0)
        def _():
            ex1a.start()

        @pl.when(jnp.logical_not(keep0))
        def _():
            ex1b.start()

        ex1a.wait_recv()

        @pl.when(keep0)
        def _():
            sbuf[...] = part[0] + rbuf[0]

        @pl.when(jnp.logical_not(keep0))
        def _():
            sbuf[...] = part[1] + rbuf[0]

        ex2 = pltpu.make_async_remote_copy(
            src_ref=sbuf, dst_ref=rbuf.at[1],
            send_sem=psend.at[1], recv_sem=precv.at[1],
            device_id=(p2,), device_id_type=pl.DeviceIdType.MESH,
        )
        ex2.start()
        ex2.wait_recv()
        tbuf[...] = sbuf[...] + rbuf[1]

        ex3 = pltpu.make_async_remote_copy(
            src_ref=tbuf, dst_ref=rbuf.at[2],
            send_sem=psend.at[2], recv_sem=precv.at[2],
            device_id=(p1,), device_id_type=pl.DeviceIdType.MESH,
        )
        ex3.start()
        ex3.wait_recv()

        @pl.when(keep0)
        def _():
            out_ref[0, :, :] = tbuf[...]
            out_ref[1, :, :] = rbuf[2]

        @pl.when(jnp.logical_not(keep0))
        def _():
            out_ref[0, :, :] = rbuf[2]
            out_ref[1, :, :] = tbuf[...]

        ex1a.wait_send()
        ex2.wait_send()
        ex3.wait_send()

        @pl.when(my == 0)
        def _():
            for r in kv_rdmas:
                r.wait_send()

    return pl.pallas_call(
        body,
        out_shape=jax.ShapeDtypeStruct((B, Sq, Do), jnp.float32),
        in_specs=[pl.BlockSpec(memory_space=pltpu.VMEM)] * 5,
        out_specs=pl.BlockSpec(memory_space=pltpu.VMEM),
        scratch_shapes=[
            pltpu.VMEM((B, Skv_l, Hl, Dh), jnp.float32),
            pltpu.VMEM((B, Skv_l, Hl, Dh), jnp.float32),
            pltpu.VMEM((B, Sq, Do), jnp.float32),
            pltpu.VMEM((3, Sq, Do), jnp.float32),
            pltpu.VMEM((Sq, Do), jnp.float32),
            pltpu.VMEM((Sq, Do), jnp.float32),
            pltpu.SemaphoreType.DMA((N_DEV - 1,)),
            pltpu.SemaphoreType.DMA((N_DEV - 1,)),
            pltpu.SemaphoreType.DMA,
            pltpu.SemaphoreType.DMA,
            pltpu.SemaphoreType.DMA((N_DEV - 1,)),
            pltpu.SemaphoreType.DMA((N_DEV - 1,)),
        ],
        compiler_params=pltpu.CompilerParams(collective_id=0),
    )(x, Wq, K_ext, V_ext, Wo)
